# Initial kernel scaffold; baseline (speedup 1.0000x reference)
import jax
import jax.numpy as jnp
from jax.experimental import pallas as pl


def kernel(vision_features, text_features, Wv, bv, Wt, bt, W1, b1, W2, b2):
    raise NotImplementedError("write your pallas kernel here")


# speedup vs baseline: 1.0835x; 1.0835x over previous
import jax
import jax.numpy as jnp
from jax.experimental import pallas as pl
from jax.experimental.pallas import tpu as pltpu

_B = 16384
_H = 4096
_RH = 256
_E = 16
_ROWS = 512

_DN = (((1,), (0,)), ((), ()))
_DN_T = (((1,), (1,)), ((), ()))


def _dot(a, b):
    return jax.lax.dot_general(a, b, _DN, precision="default",
                               preferred_element_type=jnp.float32)


def _proj(w, x):
    return jax.lax.dot_general(w, x, _DN_T, precision="default",
                               preferred_element_type=jnp.float32).T


def _router_body(xv, xt, wv, wt, bv, bt, w1t, b1, w2t, b2,
                 rw_o, tkw_o, tki_o, logits_o):
    v = _proj(wv[...], xv[...]) + bv[...]
    t = _proj(wt[...], xt[...]) + bt[...]
    combined = jnp.concatenate([v, t], axis=-1)
    pre = _dot(combined, w1t[...]) + b1[...]
    h = 0.5 * pre * (1.0 + jax.lax.erf(pre * 0.7071067811865476))
    logits = _dot(h, w2t[...]) + b2[...]

    m = jnp.max(logits, axis=-1, keepdims=True)
    e = jnp.exp(logits - m)
    ps = [e[:, i:i + 1] for i in range(_E)]
    n = _E // 2
    while n >= 1:
        ps = [ps[i] + ps[i + n] for i in range(n)]
        n //= 2
    rw = e / ps[0]

    idx = jax.lax.broadcasted_iota(jnp.int32, (_ROWS, _E), 1)
    m1 = jnp.max(rw, axis=-1, keepdims=True)
    i1 = jnp.min(jnp.where(rw == m1, idx, _E), axis=-1, keepdims=True)
    rw_masked = jnp.where(idx == i1, -1.0, rw)
    m2 = jnp.max(rw_masked, axis=-1, keepdims=True)
    i2 = jnp.min(jnp.where(rw_masked == m2, idx, _E), axis=-1, keepdims=True)

    denom = m1 + m2 + 1e-10
    rw_o[...] = rw
    tkw_o[...] = jnp.concatenate([m1 / denom, m2 / denom], axis=-1)
    tki_o[...] = jnp.concatenate([i1, i2], axis=-1)
    logits_o[...] = logits


def kernel(vision_features, text_features, Wv, bv, Wt, bt, W1, b1, W2, b2):
    w1t = W1.T
    w2t = W2.T
    bv2 = bv.reshape(1, _RH)
    bt2 = bt.reshape(1, _RH)
    b12 = b1.reshape(1, _RH)
    b22 = b2.reshape(1, _E)

    nb = _B // _ROWS
    row_spec = lambda shape: pl.BlockSpec(shape, lambda i: (i, 0))
    full_spec = lambda shape: pl.BlockSpec(shape, lambda i: (0, 0))

    return pl.pallas_call(
        _router_body,
        grid=(nb,),
        in_specs=[
            row_spec((_ROWS, _H)),
            row_spec((_ROWS, _H)),
            full_spec((_RH, _H)),
            full_spec((_RH, _H)),
            full_spec((1, _RH)),
            full_spec((1, _RH)),
            full_spec((2 * _RH, _RH)),
            full_spec((1, _RH)),
            full_spec((_RH, _E)),
            full_spec((1, _E)),
        ],
        out_specs=[
            row_spec((_ROWS, _E)),
            row_spec((_ROWS, 2)),
            row_spec((_ROWS, 2)),
            row_spec((_ROWS, _E)),
        ],
        out_shape=[
            jax.ShapeDtypeStruct((_B, _E), jnp.float32),
            jax.ShapeDtypeStruct((_B, 2), jnp.float32),
            jax.ShapeDtypeStruct((_B, 2), jnp.int32),
            jax.ShapeDtypeStruct((_B, _E), jnp.float32),
        ],
        compiler_params=pltpu.CompilerParams(
            dimension_semantics=("parallel",),
        ),
    )(vision_features, text_features, Wv, Wt, bv2, bt2,
      w1t, b12, w2t, b22)
